# Initial kernel scaffold; baseline (speedup 1.0000x reference)
import jax
import jax.numpy as jnp
from jax.experimental import pallas as pl


def kernel(receptor_x, receptor_edge_index, receptor_batch, ligand_x, ligand_edge_index, ligand_batch, emb, W1, b1, W2, b2, Wfc, bfc):
    raise NotImplementedError("write your pallas kernel here")


# speedup vs baseline: 38.7939x; 38.7939x over previous
import functools

import jax
import jax.numpy as jnp
from jax import lax
from jax.experimental import pallas as pl
import jax.experimental.pallas.tpu as pltpu
from jax.experimental.pallas import tpu_sc as plsc

N = 100000
E = 3200000
G = 64
D = 16
NCLS = 20

NSUB = 16
N_PAD = 100352
NCHUNK = N_PAD // NSUB
E_PAD = 3276800
EROWS = E_PAD // 128
ROWS_PT = EROWS // NSUB
KROW = 8
NSTEP = ROWS_PT // KROW
NOCH = 16
OCHUNK = NCHUNK // NOCH

_mesh = plsc.VectorSubcoreMesh(core_axis_name="c", subcore_axis_name="s")
_sc_params = pltpu.CompilerParams(use_tc_tiling_on_sc=False)


def _deg_body(dst3, zeros_n, ones_h, deg_out, deg_sh, dbuf, onesv, obuf, sem):
    c = lax.axis_index("c")
    s = lax.axis_index("s")
    pltpu.sync_copy(zeros_n.at[pl.ds(s * NCHUNK, NCHUNK)],
                    deg_sh.at[pl.ds(s * NCHUNK, NCHUNK)])
    pltpu.sync_copy(ones_h, onesv)
    plsc.subcore_barrier()
    row0 = s * ROWS_PT

    def step(j, carry):
        pltpu.sync_copy(dst3.at[c, pl.ds(row0 + j * KROW, KROW)], dbuf)
        for t in range(KROW):
            pltpu.sync_copy(onesv, deg_sh.at[dbuf.at[t]], add=True)
        return carry

    lax.fori_loop(0, NSTEP, step, 0)
    plsc.subcore_barrier()
    pltpu.sync_copy(deg_sh.at[pl.ds(s * NCHUNK, NCHUNK)], obuf)
    pltpu.sync_copy(obuf, deg_out.at[c, pl.ds(s * NCHUNK, NCHUNK)])


@jax.jit
def _deg_pass(dst3, zeros_n, ones_h):
    return pl.kernel(
        _deg_body,
        out_type=jax.ShapeDtypeStruct((2, N_PAD), jnp.float32),
        mesh=_mesh,
        scratch_types=[
            pltpu.VMEM_SHARED((N_PAD,), jnp.float32),
            pltpu.VMEM((KROW, 128), jnp.int32),
            pltpu.VMEM((128,), jnp.float32),
            pltpu.VMEM((NCHUNK,), jnp.float32),
            pltpu.SemaphoreType.DMA,
        ],
        compiler_params=_sc_params,
    )(dst3, zeros_n, ones_h)


def _ns_body(ytab, srcoff3, dst3, zeros16, ns_out,
             ns_sh, sbuf, dbuf, rowbuf, obuf, sem):
    c = lax.axis_index("c")
    s = lax.axis_index("s")
    pltpu.sync_copy(zeros16.at[pl.ds(s * NCHUNK, NCHUNK)],
                    ns_sh.at[pl.ds(s * NCHUNK, NCHUNK)])
    plsc.subcore_barrier()
    row0 = s * ROWS_PT

    def step(j, carry):
        pltpu.sync_copy(srcoff3.at[c, pl.ds(row0 + j * KROW, KROW)], sbuf)
        cps = [
            pltpu.async_copy(ytab.at[sbuf.at[t]],
                             rowbuf.at[pl.ds(t * 128, 128)], sem)
            for t in range(KROW)
        ]
        pltpu.sync_copy(dst3.at[c, pl.ds(row0 + j * KROW, KROW)], dbuf)
        for cp in cps:
            cp.wait()
        for t in range(KROW):
            pltpu.sync_copy(rowbuf.at[pl.ds(t * 128, 128)],
                            ns_sh.at[dbuf.at[t]], add=True)
        return carry

    lax.fori_loop(0, NSTEP, step, 0)
    plsc.subcore_barrier()
    for t in range(NOCH):
        pltpu.sync_copy(ns_sh.at[pl.ds(s * NCHUNK + t * OCHUNK, OCHUNK)], obuf)
        pltpu.sync_copy(
            obuf, ns_out.at[pl.ds(c * N_PAD + s * NCHUNK + t * OCHUNK, OCHUNK)])


@jax.jit
def _ns_pass(ytab, srcoff3, dst3, zeros16):
    return pl.kernel(
        _ns_body,
        out_type=jax.ShapeDtypeStruct((2 * N_PAD, D), jnp.float32),
        mesh=_mesh,
        scratch_types=[
            pltpu.VMEM_SHARED((N_PAD, D), jnp.float32),
            pltpu.VMEM((KROW, 128), jnp.int32),
            pltpu.VMEM((KROW, 128), jnp.int32),
            pltpu.VMEM((KROW * 128, D), jnp.float32),
            pltpu.VMEM((OCHUNK, D), jnp.float32),
            pltpu.SemaphoreType.DMA,
        ],
        compiler_params=_sc_params,
    )(ytab, srcoff3, dst3, zeros16)


BLK = 1024
NBLK = N_PAD // BLK


def _y_body(ohs_ref, emb_ref, w1_ref, y_ref):
    a1 = jnp.dot(emb_ref[...], w1_ref[...], preferred_element_type=jnp.float32, precision=lax.Precision.HIGHEST)
    y_ref[...] = jnp.dot(ohs_ref[...], a1, preferred_element_type=jnp.float32, precision=lax.Precision.HIGHEST)


@jax.jit
def _tc_y(ohs, emb, w1):
    return pl.pallas_call(
        _y_body,
        grid=(NBLK,),
        in_specs=[
            pl.BlockSpec((BLK, NCLS), lambda i: (i, 0)),
            pl.BlockSpec((NCLS, D), lambda i: (0, 0)),
            pl.BlockSpec((D, D), lambda i: (0, 0)),
        ],
        out_specs=pl.BlockSpec((BLK, D), lambda i: (i, 0)),
        out_shape=jax.ShapeDtypeStruct((N_PAD, D), jnp.float32),
    )(ohs, emb, w1)


def _z_body(ns_ref, y_ref, disrow_ref, b1_ref, z_ref):
    out1 = disrow_ref[...] * (ns_ref[...] + y_ref[...]) + b1_ref[...]
    z_ref[...] = disrow_ref[...] * jnp.maximum(out1, 0.0)


@jax.jit
def _tc_z(ns, y, disrow, b1):
    return pl.pallas_call(
        _z_body,
        grid=(NBLK,),
        in_specs=[
            pl.BlockSpec((BLK, D), lambda i: (i, 0)),
            pl.BlockSpec((BLK, D), lambda i: (i, 0)),
            pl.BlockSpec((BLK, D), lambda i: (i, 0)),
            pl.BlockSpec((1, D), lambda i: (0, 0)),
        ],
        out_specs=pl.BlockSpec((BLK, D), lambda i: (i, 0)),
        out_shape=jax.ShapeDtypeStruct((N_PAD, D), jnp.float32),
    )(ns, y, disrow, b1)


def _pool_body(ns2_ref, z_ref, disrow_ref, ohb_ref, p_ref, cnt_ref):
    @pl.when(pl.program_id(0) == 0)
    def _():
        p_ref[...] = jnp.zeros_like(p_ref)
        cnt_ref[...] = jnp.zeros_like(cnt_ref)

    m = disrow_ref[...] * (ns2_ref[...] + z_ref[...])
    ohb = ohb_ref[...]
    dn = (((0,), (0,)), ((), ()))
    p_ref[...] += lax.dot_general(ohb, m, dn, preferred_element_type=jnp.float32, precision=lax.Precision.HIGHEST)
    cnt_ref[...] += lax.dot_general(
        ohb, jnp.ones_like(m), dn, preferred_element_type=jnp.float32, precision=lax.Precision.HIGHEST)


@jax.jit
def _tc_pool(ns2, z, disrow, ohb):
    return pl.pallas_call(
        _pool_body,
        grid=(NBLK,),
        in_specs=[
            pl.BlockSpec((BLK, D), lambda i: (i, 0)),
            pl.BlockSpec((BLK, D), lambda i: (i, 0)),
            pl.BlockSpec((BLK, D), lambda i: (i, 0)),
            pl.BlockSpec((BLK, G), lambda i: (i, 0)),
        ],
        out_specs=[
            pl.BlockSpec((G, D), lambda i: (0, 0)),
            pl.BlockSpec((G, D), lambda i: (0, 0)),
        ],
        out_shape=[
            jax.ShapeDtypeStruct((G, D), jnp.float32),
            jax.ShapeDtypeStruct((G, D), jnp.float32),
        ],
    )(ns2, z, disrow, ohb)


def _fin_body(pr_ref, pl_ref, cr_ref, cl_ref, w2_ref, b2_ref, wfc_ref, bfc_ref,
              out_ref):
    w2 = w2_ref[...]
    pooled_r = (jnp.dot(pr_ref[...], w2, preferred_element_type=jnp.float32, precision=lax.Precision.HIGHEST)
                / jnp.maximum(cr_ref[...], 1.0)) + b2_ref[...]
    pooled_l = (jnp.dot(pl_ref[...], w2, preferred_element_type=jnp.float32, precision=lax.Precision.HIGHEST)
                / jnp.maximum(cl_ref[...], 1.0)) + b2_ref[...]
    h = jnp.concatenate([pooled_r, pooled_l], axis=1)
    out_ref[...] = jnp.dot(h, wfc_ref[...],
                           preferred_element_type=jnp.float32, precision=lax.Precision.HIGHEST) + bfc_ref[...]


@jax.jit
def _tc_fin(p_r, p_l, c_r, c_l, w2, b2, wfc, bfc):
    return pl.pallas_call(
        _fin_body,
        out_shape=jax.ShapeDtypeStruct((G, 6), jnp.float32),
    )(p_r, p_l, c_r, c_l, w2, b2, wfc, bfc)


@jax.jit
def kernel(receptor_x, receptor_edge_index, receptor_batch,
           ligand_x, ligand_edge_index, ligand_batch,
           emb, W1, b1, W2, b2, Wfc, bfc):
    f32 = jnp.float32

    def prep_edges(ei):
        src = ei[0].astype(jnp.int32)
        dst = ei[1].astype(jnp.int32)
        src = jnp.pad(src, (0, E_PAD - E))
        dst = jnp.pad(dst, (0, E_PAD - E), constant_values=N_PAD - 1)
        return src, dst

    rs, rd = prep_edges(receptor_edge_index)
    ls, ld = prep_edges(ligand_edge_index)
    srcoff3 = jnp.stack([rs, ls + N_PAD]).reshape(2, EROWS, 128)
    dst3 = jnp.stack([rd, ld]).reshape(2, EROWS, 128)

    zeros_n = jnp.zeros((N_PAD,), f32)
    zeros16 = jnp.zeros((N_PAD, D), f32)
    ones_h = jnp.ones((128,), f32)

    indeg2 = _deg_pass(dst3, zeros_n, ones_h)
    dis2 = lax.rsqrt(indeg2 + 1.0)

    cls20 = jnp.arange(NCLS, dtype=jnp.int32)
    gid = jnp.arange(G, dtype=jnp.int32)

    def encode(x_idx, batch, dis):
        xp = jnp.pad(x_idx.astype(jnp.int32), (0, N_PAD - N))
        ohs = (xp[:, None] == cls20[None, :]).astype(f32) * dis[:, None]
        bp = jnp.pad(batch.astype(jnp.int32), (0, N_PAD - N),
                     constant_values=-1)
        ohb = (bp[:, None] == gid[None, :]).astype(f32)
        disrow = jnp.broadcast_to(dis[:, None], (N_PAD, D))
        return ohs, ohb, disrow

    ohs_r, ohb_r, disrow_r = encode(receptor_x, receptor_batch, dis2[0])
    ohs_l, ohb_l, disrow_l = encode(ligand_x, ligand_batch, dis2[1])

    y_r = _tc_y(ohs_r, emb, W1)
    y_l = _tc_y(ohs_l, emb, W1)
    ytab = jnp.concatenate([y_r, y_l], axis=0)

    ns1 = _ns_pass(ytab, srcoff3, dst3, zeros16)

    b1r = b1.reshape(1, D)
    z_r = _tc_z(ns1[:N_PAD], y_r, disrow_r, b1r)
    z_l = _tc_z(ns1[N_PAD:], y_l, disrow_l, b1r)
    ztab = jnp.concatenate([z_r, z_l], axis=0)

    ns2 = _ns_pass(ztab, srcoff3, dst3, zeros16)

    p_r, c_r = _tc_pool(ns2[:N_PAD], z_r, disrow_r, ohb_r)
    p_l, c_l = _tc_pool(ns2[N_PAD:], z_l, disrow_l, ohb_l)

    return _tc_fin(p_r, p_l, c_r, c_l, W2, b2.reshape(1, D), Wfc,
                   bfc.reshape(1, 6))


# speedup vs baseline: 45.8187x; 1.1811x over previous
import jax
import jax.numpy as jnp
from jax import lax
from jax.experimental import pallas as pl
import jax.experimental.pallas.tpu as pltpu
from jax.experimental.pallas import tpu_sc as plsc

N = 100000
E = 3200000
G = 64
D = 16
NCLS = 20

NSUB = 16
N_PAD = 100352
NCHUNK = N_PAD // NSUB
E_PAD = 3276800
EROWS = E_PAD // 128
ROWS_PT = EROWS // NSUB
NSC = 16
SCHUNK = NCHUNK // NSC

_mesh = plsc.VectorSubcoreMesh(core_axis_name="c", subcore_axis_name="s")
_sc_params = pltpu.CompilerParams(use_tc_tiling_on_sc=False,
                                  needs_layout_passes=False)

def _rsqrt16(x):
    i = lax.bitcast_convert_type(x, jnp.int32)
    i = jnp.int32(0x5F3759DF) - jnp.right_shift(i, 1)
    y = lax.bitcast_convert_type(i, jnp.float32)
    hx = 0.5 * x
    for _ in range(3):
        y = y * (1.5 - hx * y * y)
    return y


def _iota16():
    return jax.lax.iota(jnp.int32, 16)


P1_KROW = 8
P1_NSTEP = ROWS_PT // P1_KROW


def _p1_body(dst3, xp2, a1, zeros_n, ones_h,
             ytab_out, dis_out,
             deg_sh, dbuf, onesv, degbuf, xbuf, disbuf, a1buf, ybuf, sem):
    c = lax.axis_index("c")
    s = lax.axis_index("s")
    pltpu.sync_copy(zeros_n.at[pl.ds(s * NCHUNK, NCHUNK)],
                    deg_sh.at[pl.ds(s * NCHUNK, NCHUNK)])
    pltpu.sync_copy(ones_h, onesv)
    pltpu.sync_copy(a1, a1buf)
    pltpu.sync_copy(xp2.at[c, pl.ds(s * NCHUNK, NCHUNK)], xbuf)
    plsc.subcore_barrier()
    row0 = s * ROWS_PT

    def step(j, carry):
        pltpu.sync_copy(dst3.at[c, pl.ds(row0 + j * P1_KROW, P1_KROW)], dbuf)
        for t in range(P1_KROW):
            pltpu.sync_copy(onesv, deg_sh.at[dbuf.at[t]], add=True)
        return carry

    lax.fori_loop(0, P1_NSTEP, step, 0)
    plsc.subcore_barrier()

    pltpu.sync_copy(deg_sh.at[pl.ds(s * NCHUNK, NCHUNK)], degbuf)

    def dstep(k, carry):
        degv = degbuf[pl.ds(k * 16, 16)]
        disbuf[pl.ds(k * 16, 16)] = _rsqrt16(degv + 1.0)
        return carry

    lax.fori_loop(0, NCHUNK // 16, dstep, 0)
    pltpu.sync_copy(disbuf, dis_out.at[c, pl.ds(s * NCHUNK, NCHUNK)])

    iota = _iota16()
    for sub in range(NSC):
        base_n = sub * SCHUNK

        def ystep(i, carry):
            idx16 = jnp.full((16,), base_n + i, jnp.int32)
            cls = plsc.load_gather(xbuf, [idx16])
            dsv = plsc.load_gather(disbuf, [idx16])
            row = plsc.load_gather(a1buf, [cls * 16 + iota])
            ybuf[i] = row * dsv
            return carry

        lax.fori_loop(0, SCHUNK, ystep, 0)
        pltpu.sync_copy(
            ybuf, ytab_out.at[pl.ds(c * N_PAD + s * NCHUNK + base_n, SCHUNK)])


@jax.jit
def _p1_pass(dst3, xp2, a1, zeros_n, ones_h):
    return pl.kernel(
        _p1_body,
        out_type=(
            jax.ShapeDtypeStruct((2 * N_PAD, D), jnp.float32),
            jax.ShapeDtypeStruct((2, N_PAD), jnp.float32),
        ),
        mesh=_mesh,
        scratch_types=[
            pltpu.VMEM_SHARED((N_PAD,), jnp.float32),
            pltpu.VMEM((P1_KROW, 128), jnp.int32),
            pltpu.VMEM((128,), jnp.float32),
            pltpu.VMEM((NCHUNK,), jnp.float32),
            pltpu.VMEM((NCHUNK,), jnp.int32),
            pltpu.VMEM((NCHUNK,), jnp.float32),
            pltpu.VMEM((NCLS * D,), jnp.float32),
            pltpu.VMEM((SCHUNK, D), jnp.float32),
            pltpu.SemaphoreType.DMA,
        ],
        compiler_params=_sc_params,
    )(dst3, xp2, a1, zeros_n, ones_h)


KROW = 4
NSTEP = ROWS_PT // KROW


def _edge_accum(ytab, srcoff3, dst3, ns_sh, sbuf, dbuf, rowbuf, sem, c, s):
    row0 = s * ROWS_PT

    def step(j, carry):
        pltpu.sync_copy(srcoff3.at[c, pl.ds(row0 + j * KROW, KROW)], sbuf)
        cps = [
            pltpu.async_copy(ytab.at[sbuf.at[t]],
                             rowbuf.at[pl.ds(t * 128, 128)], sem)
            for t in range(KROW)
        ]
        pltpu.sync_copy(dst3.at[c, pl.ds(row0 + j * KROW, KROW)], dbuf)
        for cp in cps:
            cp.wait()
        for t in range(KROW):
            pltpu.sync_copy(rowbuf.at[pl.ds(t * 128, 128)],
                            ns_sh.at[dbuf.at[t]], add=True)
        return carry

    lax.fori_loop(0, NSTEP, step, 0)


def _p2_body(ytab, srcoff3, dst3, dis2, b1, zeros16,
             ztab_out,
             ns_sh, sbuf, dbuf, rowbuf, nsbuf, ybuf, disb, b1buf, sem):
    c = lax.axis_index("c")
    s = lax.axis_index("s")
    pltpu.sync_copy(zeros16.at[pl.ds(s * NCHUNK, NCHUNK)],
                    ns_sh.at[pl.ds(s * NCHUNK, NCHUNK)])
    pltpu.sync_copy(b1, b1buf)
    plsc.subcore_barrier()
    _edge_accum(ytab, srcoff3, dst3, ns_sh, sbuf, dbuf, rowbuf, sem, c, s)
    plsc.subcore_barrier()

    b1v = b1buf[...]
    for sub in range(NSC):
        base_n = s * NCHUNK + sub * SCHUNK
        pltpu.sync_copy(ns_sh.at[pl.ds(s * NCHUNK + sub * SCHUNK, SCHUNK)],
                        nsbuf)
        pltpu.sync_copy(ytab.at[pl.ds(c * N_PAD + base_n, SCHUNK)], ybuf)
        pltpu.sync_copy(dis2.at[c, pl.ds(base_n, SCHUNK)], disb)

        def zstep(i, carry):
            idx16 = jnp.full((16,), i, jnp.int32)
            dsv = plsc.load_gather(disb, [idx16])
            out1 = dsv * (nsbuf[i] + ybuf[i]) + b1v
            nsbuf[i] = dsv * jnp.maximum(out1, 0.0)
            return carry

        lax.fori_loop(0, SCHUNK, zstep, 0)
        pltpu.sync_copy(nsbuf, ztab_out.at[pl.ds(c * N_PAD + base_n, SCHUNK)])


@jax.jit
def _p2_pass(ytab, srcoff3, dst3, dis2, b1, zeros16):
    return pl.kernel(
        _p2_body,
        out_type=jax.ShapeDtypeStruct((2 * N_PAD, D), jnp.float32),
        mesh=_mesh,
        scratch_types=[
            pltpu.VMEM_SHARED((N_PAD, D), jnp.float32),
            pltpu.VMEM((KROW, 128), jnp.int32),
            pltpu.VMEM((KROW, 128), jnp.int32),
            pltpu.VMEM((KROW * 128, D), jnp.float32),
            pltpu.VMEM((SCHUNK, D), jnp.float32),
            pltpu.VMEM((SCHUNK, D), jnp.float32),
            pltpu.VMEM((SCHUNK,), jnp.float32),
            pltpu.VMEM((16,), jnp.float32),
            pltpu.SemaphoreType.DMA,
        ],
        compiler_params=_sc_params,
    )(ytab, srcoff3, dst3, dis2, b1, zeros16)


NBKT = 65


def _p3_body(ztab, srcoff3, dst3, dis2, bp2, zeros16,
             acc_out, cnt_out,
             ns_sh, sbuf, dbuf, rowbuf, nsbuf, zbuf, disb, batchb,
             accb, cntb, sem):
    c = lax.axis_index("c")
    s = lax.axis_index("s")
    pltpu.sync_copy(zeros16.at[pl.ds(s * NCHUNK, NCHUNK)],
                    ns_sh.at[pl.ds(s * NCHUNK, NCHUNK)])

    def zerostep(k, carry):
        accb[pl.ds(k * 16, 16)] = jnp.zeros((16,), jnp.float32)
        cntb[pl.ds(k * 16, 16)] = jnp.zeros((16,), jnp.float32)
        return carry

    lax.fori_loop(0, NBKT, zerostep, 0)
    plsc.subcore_barrier()
    _edge_accum(ztab, srcoff3, dst3, ns_sh, sbuf, dbuf, rowbuf, sem, c, s)
    plsc.subcore_barrier()

    iota = _iota16()
    ones = jnp.ones((16,), jnp.float32)
    for sub in range(NSC):
        base_n = s * NCHUNK + sub * SCHUNK
        pltpu.sync_copy(ns_sh.at[pl.ds(s * NCHUNK + sub * SCHUNK, SCHUNK)],
                        nsbuf)
        pltpu.sync_copy(ztab.at[pl.ds(c * N_PAD + base_n, SCHUNK)], zbuf)
        pltpu.sync_copy(dis2.at[c, pl.ds(base_n, SCHUNK)], disb)
        pltpu.sync_copy(bp2.at[c, pl.ds(base_n, SCHUNK)], batchb)

        def pstep(i, carry):
            idx16 = jnp.full((16,), i, jnp.int32)
            dsv = plsc.load_gather(disb, [idx16])
            b = plsc.load_gather(batchb, [idx16])
            beff = jnp.where(b < 0, G, b)
            m = dsv * (nsbuf[i] + zbuf[i])
            slot = beff * 16 + iota
            plsc.addupdate_scatter(accb, [slot], m)
            plsc.addupdate_scatter(cntb, [slot], ones)
            return carry

        lax.fori_loop(0, SCHUNK, pstep, 0)

    pltpu.sync_copy(accb, acc_out.at[c, s])
    pltpu.sync_copy(cntb, cnt_out.at[c, s])


@jax.jit
def _p3_pass(ztab, srcoff3, dst3, dis2, bp2, zeros16):
    return pl.kernel(
        _p3_body,
        out_type=(
            jax.ShapeDtypeStruct((2, NSUB, NBKT * D), jnp.float32),
            jax.ShapeDtypeStruct((2, NSUB, NBKT * D), jnp.float32),
        ),
        mesh=_mesh,
        scratch_types=[
            pltpu.VMEM_SHARED((N_PAD, D), jnp.float32),
            pltpu.VMEM((KROW, 128), jnp.int32),
            pltpu.VMEM((KROW, 128), jnp.int32),
            pltpu.VMEM((KROW * 128, D), jnp.float32),
            pltpu.VMEM((SCHUNK, D), jnp.float32),
            pltpu.VMEM((SCHUNK, D), jnp.float32),
            pltpu.VMEM((SCHUNK,), jnp.float32),
            pltpu.VMEM((SCHUNK,), jnp.int32),
            pltpu.VMEM((NBKT * D,), jnp.float32),
            pltpu.VMEM((NBKT * D,), jnp.float32),
            pltpu.SemaphoreType.DMA,
        ],
        compiler_params=_sc_params,
    )(ztab, srcoff3, dst3, dis2, bp2, zeros16)


def _a1_body(emb_ref, w1_ref, a1_ref):
    a1_ref[...] = jnp.dot(emb_ref[...], w1_ref[...],
                          preferred_element_type=jnp.float32)


@jax.jit
def _tc_a1(emb, w1):
    return pl.pallas_call(
        _a1_body,
        out_shape=jax.ShapeDtypeStruct((NCLS, D), jnp.float32),
    )(emb, w1)


def _fin_body(acc_ref, cnt_ref, w2_ref, b2_ref, wfc_ref, bfc_ref, out_ref):
    w2 = w2_ref[...].astype(jnp.bfloat16).astype(jnp.float32)
    p_r = jnp.sum(acc_ref[0], axis=0)[:G]
    p_l = jnp.sum(acc_ref[1], axis=0)[:G]
    c_r = jnp.sum(cnt_ref[0], axis=0)[:G]
    c_l = jnp.sum(cnt_ref[1], axis=0)[:G]
    pooled_r = (jnp.dot(p_r, w2, preferred_element_type=jnp.float32,
                        precision=lax.Precision.HIGHEST)
                / jnp.maximum(c_r, 1.0)) + b2_ref[...]
    pooled_l = (jnp.dot(p_l, w2, preferred_element_type=jnp.float32,
                        precision=lax.Precision.HIGHEST)
                / jnp.maximum(c_l, 1.0)) + b2_ref[...]
    h = jnp.concatenate([pooled_r, pooled_l], axis=1)
    out_ref[...] = jnp.dot(h, wfc_ref[...],
                           preferred_element_type=jnp.float32) + bfc_ref[...]


@jax.jit
def _tc_fin(acc, cnt, w2, b2, wfc, bfc):
    return pl.pallas_call(
        _fin_body,
        out_shape=jax.ShapeDtypeStruct((G, 6), jnp.float32),
    )(acc, cnt, w2, b2, wfc, bfc)


@jax.jit
def kernel(receptor_x, receptor_edge_index, receptor_batch,
           ligand_x, ligand_edge_index, ligand_batch,
           emb, W1, b1, W2, b2, Wfc, bfc):
    f32 = jnp.float32

    def prep_edges(ei):
        src = ei[0].astype(jnp.int32)
        dst = ei[1].astype(jnp.int32)
        src = jnp.pad(src, (0, E_PAD - E))
        dst = jnp.pad(dst, (0, E_PAD - E), constant_values=N_PAD - 1)
        return src, dst

    rs, rd = prep_edges(receptor_edge_index)
    ls, ld = prep_edges(ligand_edge_index)
    srcoff3 = jnp.stack([rs, ls + N_PAD]).reshape(2, EROWS, 128)
    dst3 = jnp.stack([rd, ld]).reshape(2, EROWS, 128)

    xp2 = jnp.stack([
        jnp.pad(receptor_x.astype(jnp.int32), (0, N_PAD - N)),
        jnp.pad(ligand_x.astype(jnp.int32), (0, N_PAD - N)),
    ])
    bp2 = jnp.stack([
        jnp.pad(receptor_batch.astype(jnp.int32), (0, N_PAD - N),
                constant_values=-1),
        jnp.pad(ligand_batch.astype(jnp.int32), (0, N_PAD - N),
                constant_values=-1),
    ])

    zeros_n = jnp.zeros((N_PAD,), f32)
    zeros16 = jnp.zeros((N_PAD, D), f32)
    ones_h = jnp.ones((128,), f32)

    a1 = _tc_a1(emb, W1)
    ytab, dis2 = _p1_pass(dst3, xp2, a1.reshape(-1), zeros_n, ones_h)
    ztab = _p2_pass(ytab, srcoff3, dst3, dis2, b1, zeros16)
    acc, cnt = _p3_pass(ztab, srcoff3, dst3, dis2, bp2, zeros16)
    acc = acc.reshape(2, NSUB, NBKT, D)
    cnt = cnt.reshape(2, NSUB, NBKT, D)
    return _tc_fin(acc, cnt, W2, b2.reshape(1, D), Wfc, bfc.reshape(1, 6))


# speedup vs baseline: 65.5213x; 1.4300x over previous
import jax
import jax.numpy as jnp
from jax import lax
from jax.experimental import pallas as pl
import jax.experimental.pallas.tpu as pltpu
from jax.experimental.pallas import tpu_sc as plsc

N = 100000
E = 3200000
G = 64
D = 16
NCLS = 20

NSUB = 16
N_PAD = 100352
NCHUNK = N_PAD // NSUB
E_PAD = 3276800
EROWS = E_PAD // 128
ROWS_PT = EROWS // NSUB
NSC = 28
SCHUNK = NCHUNK // NSC

_mesh = plsc.VectorSubcoreMesh(core_axis_name="c", subcore_axis_name="s")
_sc_params = pltpu.CompilerParams(use_tc_tiling_on_sc=False,
                                  needs_layout_passes=False)

def _rsqrt16(x):
    i = lax.bitcast_convert_type(x, jnp.int32)
    i = jnp.int32(0x5F3759DF) - jnp.right_shift(i, 1)
    y = lax.bitcast_convert_type(i, jnp.float32)
    hx = 0.5 * x
    for _ in range(3):
        y = y * (1.5 - hx * y * y)
    return y


def _iota16():
    return jax.lax.iota(jnp.int32, 16)


P1_KROW = 8
P1_NSTEP = ROWS_PT // P1_KROW


def _p1_body(dst3, xp2, a1, zeros_n, ones_h,
             ytab_out, dis_out,
             deg_sh, dbuf, onesv, degbuf, xbuf, disbuf, a1buf, ybuf, sem):
    c = lax.axis_index("c")
    s = lax.axis_index("s")
    pltpu.sync_copy(zeros_n.at[pl.ds(s * NCHUNK, NCHUNK)],
                    deg_sh.at[pl.ds(s * NCHUNK, NCHUNK)])
    pltpu.sync_copy(ones_h, onesv)
    pltpu.sync_copy(a1, a1buf)
    pltpu.sync_copy(xp2.at[c, pl.ds(s * NCHUNK, NCHUNK)], xbuf)
    plsc.subcore_barrier()
    row0 = s * ROWS_PT

    def step(j, carry):
        pltpu.sync_copy(dst3.at[c, pl.ds(row0 + j * P1_KROW, P1_KROW)], dbuf)
        for t in range(P1_KROW):
            pltpu.sync_copy(onesv, deg_sh.at[dbuf.at[t]], add=True)
        return carry

    lax.fori_loop(0, P1_NSTEP, step, 0)
    plsc.subcore_barrier()

    pltpu.sync_copy(deg_sh.at[pl.ds(s * NCHUNK, NCHUNK)], degbuf)

    def dstep(k, carry):
        degv = degbuf[pl.ds(k * 16, 16)]
        disbuf[pl.ds(k * 16, 16)] = _rsqrt16(degv + 1.0)
        return carry

    lax.fori_loop(0, NCHUNK // 16, dstep, 0)
    pltpu.sync_copy(disbuf, dis_out.at[c, pl.ds(s * NCHUNK, NCHUNK)])

    iota = _iota16()
    for sub in range(NSC):
        base_n = sub * SCHUNK

        def ystep(i, carry):
            idx16 = jnp.full((16,), base_n + i, jnp.int32)
            cls = plsc.load_gather(xbuf, [idx16])
            dsv = plsc.load_gather(disbuf, [idx16])
            row = plsc.load_gather(a1buf, [cls * 16 + iota])
            ybuf[i] = row * dsv
            return carry

        lax.fori_loop(0, SCHUNK, ystep, 0)
        pltpu.sync_copy(
            ybuf, ytab_out.at[pl.ds(c * N_PAD + s * NCHUNK + base_n, SCHUNK)])


@jax.jit
def _p1_pass(dst3, xp2, a1, zeros_n, ones_h):
    return pl.kernel(
        _p1_body,
        out_type=(
            jax.ShapeDtypeStruct((2 * N_PAD, D), jnp.float32),
            jax.ShapeDtypeStruct((2, N_PAD), jnp.float32),
        ),
        mesh=_mesh,
        scratch_types=[
            pltpu.VMEM_SHARED((N_PAD,), jnp.float32),
            pltpu.VMEM((P1_KROW, 128), jnp.int32),
            pltpu.VMEM((128,), jnp.float32),
            pltpu.VMEM((NCHUNK,), jnp.float32),
            pltpu.VMEM((NCHUNK,), jnp.int32),
            pltpu.VMEM((NCHUNK,), jnp.float32),
            pltpu.VMEM((NCLS * D,), jnp.float32),
            pltpu.VMEM((SCHUNK, D), jnp.float32),
            pltpu.SemaphoreType.DMA,
        ],
        compiler_params=_sc_params,
    )(dst3, xp2, a1, zeros_n, ones_h)


KROW = 4
NSTEP = ROWS_PT // KROW


def _edge_accum(ytab, edges3, ns_sh, ebufs, rowbufs, gsems, ssems, c, s):
    row0 = s * ROWS_PT

    def idx_dma(p, j):
        pltpu.sync_copy(edges3.at[c, pl.ds(2 * (row0 + j * KROW), 2 * KROW)],
                        ebufs[p])

    def g_fire(p):
        for t in range(KROW):
            pltpu.async_copy(ytab.at[ebufs[p].at[2 * t]],
                             rowbufs[p].at[pl.ds(t * 128, 128)], gsems[p])

    def g_wait(p):
        for t in range(KROW):
            pltpu.make_async_copy(ytab.at[ebufs[p].at[2 * t]],
                                  rowbufs[p].at[pl.ds(t * 128, 128)],
                                  gsems[p]).wait()

    def s_fire(p):
        for t in range(KROW):
            pltpu.async_copy(rowbufs[p].at[pl.ds(t * 128, 128)],
                             ns_sh.at[ebufs[p].at[2 * t + 1]], ssems[p], add=True)

    def s_wait(p):
        for t in range(KROW):
            pltpu.make_async_copy(rowbufs[p].at[pl.ds(t * 128, 128)],
                                  ns_sh.at[ebufs[p].at[2 * t + 1]],
                                  ssems[p]).wait()

    idx_dma(0, 0)
    g_fire(0)
    idx_dma(1, 1)
    g_fire(1)
    g_wait(0)
    s_fire(0)
    s_wait(0)
    idx_dma(0, 2)
    g_fire(0)
    g_wait(1)
    s_fire(1)

    def body(k, carry):
        s_wait(1)
        idx_dma(1, 2 * k + 1)
        g_fire(1)
        g_wait(0)
        s_fire(0)
        s_wait(0)
        idx_dma(0, 2 * k + 2)
        g_fire(0)
        g_wait(1)
        s_fire(1)
        return carry

    lax.fori_loop(1, NSTEP // 2 - 1, body, 0)

    s_wait(1)
    idx_dma(1, NSTEP - 1)
    g_fire(1)
    g_wait(0)
    s_fire(0)
    s_wait(0)
    g_wait(1)
    s_fire(1)
    s_wait(1)


def _p2_body(ytab, edges3, dis2, b1, zeros16,
             ztab_out,
             ns_sh, ebuf0, ebuf1, rowbuf0, rowbuf1, nsbuf, ybuf, disb, b1buf,
             gsem0, gsem1, ssem0, ssem1):
    c = lax.axis_index("c")
    s = lax.axis_index("s")
    pltpu.sync_copy(zeros16.at[pl.ds(s * NCHUNK, NCHUNK)],
                    ns_sh.at[pl.ds(s * NCHUNK, NCHUNK)])
    pltpu.sync_copy(b1, b1buf)
    plsc.subcore_barrier()
    _edge_accum(ytab, edges3, ns_sh, (ebuf0, ebuf1), (rowbuf0, rowbuf1),
                (gsem0, gsem1), (ssem0, ssem1), c, s)
    plsc.subcore_barrier()

    b1v = b1buf[...]
    for sub in range(NSC):
        base_n = s * NCHUNK + sub * SCHUNK
        pltpu.sync_copy(ns_sh.at[pl.ds(s * NCHUNK + sub * SCHUNK, SCHUNK)],
                        nsbuf)
        pltpu.sync_copy(ytab.at[pl.ds(c * N_PAD + base_n, SCHUNK)], ybuf)
        pltpu.sync_copy(dis2.at[c, pl.ds(base_n, SCHUNK)], disb)

        def zstep(i, carry):
            idx16 = jnp.full((16,), i, jnp.int32)
            dsv = plsc.load_gather(disb, [idx16])
            out1 = dsv * (nsbuf[i] + ybuf[i]) + b1v
            nsbuf[i] = dsv * jnp.maximum(out1, 0.0)
            return carry

        lax.fori_loop(0, SCHUNK, zstep, 0)
        pltpu.sync_copy(nsbuf, ztab_out.at[pl.ds(c * N_PAD + base_n, SCHUNK)])


@jax.jit
def _p2_pass(ytab, edges3, dis2, b1, zeros16):
    return pl.kernel(
        _p2_body,
        out_type=jax.ShapeDtypeStruct((2 * N_PAD, D), jnp.float32),
        mesh=_mesh,
        scratch_types=[
            pltpu.VMEM_SHARED((N_PAD, D), jnp.float32),
            pltpu.VMEM((2 * KROW, 128), jnp.int32),
            pltpu.VMEM((2 * KROW, 128), jnp.int32),
            pltpu.VMEM((KROW * 128, D), jnp.float32),
            pltpu.VMEM((KROW * 128, D), jnp.float32),
            pltpu.VMEM((SCHUNK, D), jnp.float32),
            pltpu.VMEM((SCHUNK, D), jnp.float32),
            pltpu.VMEM((SCHUNK,), jnp.float32),
            pltpu.VMEM((16,), jnp.float32),
            pltpu.SemaphoreType.DMA,
            pltpu.SemaphoreType.DMA,
            pltpu.SemaphoreType.DMA,
            pltpu.SemaphoreType.DMA,
        ],
        compiler_params=_sc_params,
    )(ytab, edges3, dis2, b1, zeros16)


NBKT = 65


def _p3_body(ztab, edges3, dis2, bp2, zeros16,
             acc_out, cnt_out,
             ns_sh, ebuf0, ebuf1, rowbuf0, rowbuf1, nsbuf, zbuf, disb, batchb,
             accb, cntb, gsem0, gsem1, ssem0, ssem1):
    c = lax.axis_index("c")
    s = lax.axis_index("s")
    pltpu.sync_copy(zeros16.at[pl.ds(s * NCHUNK, NCHUNK)],
                    ns_sh.at[pl.ds(s * NCHUNK, NCHUNK)])

    def zerostep(k, carry):
        accb[pl.ds(k * 16, 16)] = jnp.zeros((16,), jnp.float32)
        cntb[pl.ds(k * 16, 16)] = jnp.zeros((16,), jnp.float32)
        return carry

    lax.fori_loop(0, NBKT, zerostep, 0)
    plsc.subcore_barrier()
    _edge_accum(ztab, edges3, ns_sh, (ebuf0, ebuf1), (rowbuf0, rowbuf1),
                (gsem0, gsem1), (ssem0, ssem1), c, s)
    plsc.subcore_barrier()

    iota = _iota16()
    ones = jnp.ones((16,), jnp.float32)
    for sub in range(NSC):
        base_n = s * NCHUNK + sub * SCHUNK
        pltpu.sync_copy(ns_sh.at[pl.ds(s * NCHUNK + sub * SCHUNK, SCHUNK)],
                        nsbuf)
        pltpu.sync_copy(ztab.at[pl.ds(c * N_PAD + base_n, SCHUNK)], zbuf)
        pltpu.sync_copy(dis2.at[c, pl.ds(base_n, SCHUNK)], disb)
        pltpu.sync_copy(bp2.at[c, pl.ds(base_n, SCHUNK)], batchb)

        def pstep(i, carry):
            idx16 = jnp.full((16,), i, jnp.int32)
            dsv = plsc.load_gather(disb, [idx16])
            b = plsc.load_gather(batchb, [idx16])
            beff = jnp.where(b < 0, G, b)
            m = dsv * (nsbuf[i] + zbuf[i])
            slot = beff * 16 + iota
            plsc.addupdate_scatter(accb, [slot], m)
            plsc.addupdate_scatter(cntb, [slot], ones)
            return carry

        lax.fori_loop(0, SCHUNK, pstep, 0)

    pltpu.sync_copy(accb, acc_out.at[c, s])
    pltpu.sync_copy(cntb, cnt_out.at[c, s])


@jax.jit
def _p3_pass(ztab, edges3, dis2, bp2, zeros16):
    return pl.kernel(
        _p3_body,
        out_type=(
            jax.ShapeDtypeStruct((2, NSUB, NBKT * D), jnp.float32),
            jax.ShapeDtypeStruct((2, NSUB, NBKT * D), jnp.float32),
        ),
        mesh=_mesh,
        scratch_types=[
            pltpu.VMEM_SHARED((N_PAD, D), jnp.float32),
            pltpu.VMEM((2 * KROW, 128), jnp.int32),
            pltpu.VMEM((2 * KROW, 128), jnp.int32),
            pltpu.VMEM((KROW * 128, D), jnp.float32),
            pltpu.VMEM((KROW * 128, D), jnp.float32),
            pltpu.VMEM((SCHUNK, D), jnp.float32),
            pltpu.VMEM((SCHUNK, D), jnp.float32),
            pltpu.VMEM((SCHUNK,), jnp.float32),
            pltpu.VMEM((SCHUNK,), jnp.int32),
            pltpu.VMEM((NBKT * D,), jnp.float32),
            pltpu.VMEM((NBKT * D,), jnp.float32),
            pltpu.SemaphoreType.DMA,
            pltpu.SemaphoreType.DMA,
            pltpu.SemaphoreType.DMA,
            pltpu.SemaphoreType.DMA,
        ],
        compiler_params=_sc_params,
    )(ztab, edges3, dis2, bp2, zeros16)


def _a1_body(emb_ref, w1_ref, a1_ref):
    a1_ref[...] = jnp.dot(emb_ref[...], w1_ref[...],
                          preferred_element_type=jnp.float32)


@jax.jit
def _tc_a1(emb, w1):
    return pl.pallas_call(
        _a1_body,
        out_shape=jax.ShapeDtypeStruct((NCLS, D), jnp.float32),
    )(emb, w1)


def _fin_body(acc_ref, cnt_ref, w2_ref, b2_ref, wfc_ref, bfc_ref, out_ref):
    w2 = w2_ref[...].astype(jnp.bfloat16).astype(jnp.float32)
    p_r = jnp.sum(acc_ref[0], axis=0)[:G]
    p_l = jnp.sum(acc_ref[1], axis=0)[:G]
    c_r = jnp.sum(cnt_ref[0], axis=0)[:G]
    c_l = jnp.sum(cnt_ref[1], axis=0)[:G]
    pooled_r = (jnp.dot(p_r, w2, preferred_element_type=jnp.float32,
                        precision=lax.Precision.HIGHEST)
                / jnp.maximum(c_r, 1.0)) + b2_ref[...]
    pooled_l = (jnp.dot(p_l, w2, preferred_element_type=jnp.float32,
                        precision=lax.Precision.HIGHEST)
                / jnp.maximum(c_l, 1.0)) + b2_ref[...]
    h = jnp.concatenate([pooled_r, pooled_l], axis=1)
    out_ref[...] = jnp.dot(h, wfc_ref[...],
                           preferred_element_type=jnp.float32) + bfc_ref[...]


@jax.jit
def _tc_fin(acc, cnt, w2, b2, wfc, bfc):
    return pl.pallas_call(
        _fin_body,
        out_shape=jax.ShapeDtypeStruct((G, 6), jnp.float32),
    )(acc, cnt, w2, b2, wfc, bfc)


@jax.jit
def kernel(receptor_x, receptor_edge_index, receptor_batch,
           ligand_x, ligand_edge_index, ligand_batch,
           emb, W1, b1, W2, b2, Wfc, bfc):
    f32 = jnp.float32

    def prep_edges(ei):
        src = ei[0].astype(jnp.int32)
        dst = ei[1].astype(jnp.int32)
        src = jnp.pad(src, (0, E_PAD - E))
        dst = jnp.pad(dst, (0, E_PAD - E), constant_values=N_PAD - 1)
        return src, dst

    rs, rd = prep_edges(receptor_edge_index)
    ls, ld = prep_edges(ligand_edge_index)
    srcoff3 = jnp.stack([rs, ls + N_PAD]).reshape(2, EROWS, 128)
    dst3 = jnp.stack([rd, ld]).reshape(2, EROWS, 128)
    edges3 = jnp.stack([srcoff3, dst3], axis=2).reshape(2, 2 * EROWS, 128)

    xp2 = jnp.stack([
        jnp.pad(receptor_x.astype(jnp.int32), (0, N_PAD - N)),
        jnp.pad(ligand_x.astype(jnp.int32), (0, N_PAD - N)),
    ])
    bp2 = jnp.stack([
        jnp.pad(receptor_batch.astype(jnp.int32), (0, N_PAD - N),
                constant_values=-1),
        jnp.pad(ligand_batch.astype(jnp.int32), (0, N_PAD - N),
                constant_values=-1),
    ])

    zeros_n = jnp.zeros((N_PAD,), f32)
    zeros16 = jnp.zeros((N_PAD, D), f32)
    ones_h = jnp.ones((128,), f32)

    a1 = _tc_a1(emb, W1)
    ytab, dis2 = _p1_pass(dst3, xp2, a1.reshape(-1), zeros_n, ones_h)
    ztab = _p2_pass(ytab, edges3, dis2, b1, zeros16)
    acc, cnt = _p3_pass(ztab, edges3, dis2, bp2, zeros16)
    acc = acc.reshape(2, NSUB, NBKT, D)
    cnt = cnt.reshape(2, NSUB, NBKT, D)
    return _tc_fin(acc, cnt, W2, b2.reshape(1, D), Wfc, bfc.reshape(1, 6))
